# baseline (speedup 1.0000x reference)
import functools

import jax
import jax.numpy as jnp
from jax import lax
from jax.experimental import pallas as pl
from jax.experimental.pallas import tpu as pltpu
from jax.experimental.pallas import tpu_sc as plsc

INPUT_DIM = 3
N_FEATURES = 64
HASHMAP_SIZE = 524288
RESOLUTION = 128.0
PRIMES3 = (73856093, 19349663, 83492791)

NUM_CORES = 2
NUM_SUBCORES = 16
NUM_WORKERS = NUM_CORES * NUM_SUBCORES
LANES = 16

CHUNK_PTS = 64
ROWS_PER_CHUNK = CHUNK_PTS * 8


def _prep_body(x_ref, ids_ref, wgt_ref):
    xb = x_ref[...]
    xx = (xb + 1.0) / 2.0 * RESOLUTION
    xi = xx.astype(jnp.int32)
    xf = xx - xi.astype(jnp.float32)
    b_rows = xb.shape[0]
    xi_d = [lax.slice(xi, (0, d), (b_rows, d + 1)) for d in range(INPUT_DIM)]
    xf_d = [lax.slice(xf, (0, d), (b_rows, d + 1)) for d in range(INPUT_DIM)]
    ids_cols = []
    wgt_cols = []
    for k in range(8):
        h = None
        w = None
        for d in range(INPUT_DIM):
            bit = (k >> d) & 1
            t = (xi_d[d] + bit) * PRIMES3[d]
            h = t if h is None else h ^ t
            wd = xf_d[d] if bit else (1.0 - xf_d[d])
            w = wd if w is None else w * wd
        ids_cols.append(h & (HASHMAP_SIZE - 1))
        wgt_cols.append(w)
    ids_ref[...] = jnp.concatenate(ids_cols, axis=1)
    wgt_ref[...] = jnp.concatenate(wgt_cols, axis=1)


def _prep(x):
    n = x.shape[0]
    block = 2048
    return pl.pallas_call(
        _prep_body,
        grid=(n // block,),
        in_specs=[pl.BlockSpec((block, INPUT_DIM), lambda i: (i, jnp.int32(0)))],
        out_specs=[
            pl.BlockSpec((block, 8), lambda i: (i, jnp.int32(0))),
            pl.BlockSpec((block, 8), lambda i: (i, jnp.int32(0))),
        ],
        out_shape=[
            jax.ShapeDtypeStruct((n, 8), jnp.int32),
            jax.ShapeDtypeStruct((n, 8), jnp.float32),
        ],
    )(x)


def _sc_gather(W, ids2, wgt2, n):
    pts_per_w = n // NUM_WORKERS
    n_chunks = pts_per_w // CHUNK_PTS
    idx_rows = ROWS_PER_CHUNK // 128

    mesh = plsc.VectorSubcoreMesh(core_axis_name="c", subcore_axis_name="s")

    @functools.partial(
        pl.kernel,
        mesh=mesh,
        out_type=jax.ShapeDtypeStruct((n, N_FEATURES), jnp.float32),
        scratch_types=[
            pltpu.VMEM((ROWS_PER_CHUNK,), jnp.int32),
            pltpu.VMEM((ROWS_PER_CHUNK,), jnp.float32),
            pltpu.VMEM((ROWS_PER_CHUNK, N_FEATURES), jnp.float32),
            pltpu.VMEM((CHUNK_PTS, N_FEATURES), jnp.float32),
            pltpu.SemaphoreType.DMA,
        ],
        compiler_params=pltpu.CompilerParams(
            needs_layout_passes=False, use_tc_tiling_on_sc=False
        ),
    )
    def sc_kernel(w_hbm, ids_hbm, wgt_hbm, out_hbm, idx_v, wgt_v, rows_v, out_v, sem):
        cid = lax.axis_index("c")
        sid = lax.axis_index("s")
        wid = sid * jnp.int32(NUM_CORES) + cid

        def chunk_body(g, carry):
            pt_base = wid * jnp.int32(pts_per_w) + g * jnp.int32(CHUNK_PTS)
            fb = pt_base * jnp.int32(8)
            pltpu.sync_copy(ids_hbm.at[pl.ds(fb, ROWS_PER_CHUNK)], idx_v)
            pltpu.sync_copy(wgt_hbm.at[pl.ds(fb, ROWS_PER_CHUNK)], wgt_v)
            handles = [
                pltpu.async_copy(
                    w_hbm.at[idx_v.at[pl.ds(j * 128, 128)]],
                    rows_v.at[pl.ds(j * 128, 128)],
                    sem,
                )
                for j in range(idx_rows)
            ]
            for h in handles:
                h.wait()

            def grp_body(s4, carry2):
                row0 = s4 * jnp.int32(128)
                for i in range(16):
                    lbase = i * 8
                    wbc = [
                        plsc.load_gather(
                            wgt_v,
                            [jnp.full((LANES,), row0 + jnp.int32(lbase + k),
                                      jnp.int32)],
                        )
                        for k in range(8)
                    ]
                    r0 = row0 + lbase
                    for v in range(N_FEATURES // LANES):
                        sl = pl.ds(v * LANES, LANES)
                        acc = wbc[0] * rows_v[r0, sl]
                        for k in range(1, 8):
                            acc = acc + wbc[k] * rows_v[r0 + k, sl]
                        out_v[s4 * jnp.int32(16) + jnp.int32(i), sl] = acc
                return carry2

            lax.fori_loop(jnp.int32(0), jnp.int32(CHUNK_PTS // 16), grp_body,
                          jnp.int32(0))
            pltpu.sync_copy(out_v, out_hbm.at[pl.ds(pt_base, CHUNK_PTS)])
            return carry

        lax.fori_loop(jnp.int32(0), jnp.int32(n_chunks), chunk_body, jnp.int32(0))

    return sc_kernel(W, ids2, wgt2)


def kernel(x, W):
    n = x.shape[0]
    ids, wgt = _prep(x)
    ids2 = ids.reshape(n * 8)
    wgt2 = wgt.reshape(n * 8)
    return _sc_gather(W, ids2, wgt2, n)


# speedup vs baseline: 2.2802x; 2.2802x over previous
import functools

import jax
import jax.numpy as jnp
from jax import lax
from jax.experimental import pallas as pl
from jax.experimental.pallas import tpu as pltpu
from jax.experimental.pallas import tpu_sc as plsc

N_FEATURES = 64
HASHMAP_SIZE = 524288
RESOLUTION = 128.0
PRIMES3 = (73856093, 19349663, 83492791)

NUM_CORES = 2
NUM_SUBCORES = 16
NUM_WORKERS = NUM_CORES * NUM_SUBCORES
LANES = 16

CHUNK = 64
NV = N_FEATURES // LANES


def _sc_fused(xt, W, n):
    pts_per_w = n // NUM_WORKERS
    n_chunks = pts_per_w // CHUNK

    mesh = plsc.VectorSubcoreMesh(core_axis_name="c", subcore_axis_name="s")

    @functools.partial(
        pl.kernel,
        mesh=mesh,
        out_type=jax.ShapeDtypeStruct((n, N_FEATURES), jnp.float32),
        scratch_types=[
            pltpu.VMEM((3, pts_per_w), jnp.float32),
            pltpu.VMEM((8, CHUNK), jnp.int32),
            pltpu.VMEM((8, CHUNK), jnp.int32),
            pltpu.VMEM((8, CHUNK), jnp.float32),
            pltpu.VMEM((8, CHUNK), jnp.float32),
            pltpu.VMEM((8, CHUNK, N_FEATURES), jnp.float32),
            pltpu.VMEM((8, CHUNK, N_FEATURES), jnp.float32),
            pltpu.VMEM((CHUNK, N_FEATURES), jnp.float32),
            pltpu.VMEM((CHUNK, N_FEATURES), jnp.float32),
            pltpu.SemaphoreType.DMA,
            pltpu.SemaphoreType.DMA,
            pltpu.SemaphoreType.DMA,
            pltpu.SemaphoreType.DMA,
        ],
        compiler_params=pltpu.CompilerParams(
            needs_layout_passes=False, use_tc_tiling_on_sc=False
        ),
    )
    def sc_kernel(x_hbm, w_hbm, out_hbm, xv, idx0, idx1, wgt0, wgt1,
                  rows0, rows1, outv0, outv1, semg0, semg1, semo0, semo1):
        cid = lax.axis_index("c")
        sid = lax.axis_index("s")
        wid = sid * jnp.int32(NUM_CORES) + cid
        pt0 = wid * jnp.int32(pts_per_w)

        for d in range(3):
            pltpu.sync_copy(
                x_hbm.at[pl.ds(jnp.int32(d * n) + pt0, pts_per_w)],
                xv.at[jnp.int32(d)],
            )

        idxs = (idx0, idx1)
        wgts = (wgt0, wgt1)
        rows = (rows0, rows1)
        outs = (outv0, outv1)
        semgs = (semg0, semg1)
        semos = (semo0, semo1)

        def stage(slot, g):
            idxv, wgtv, rowsv, semg = idxs[slot], wgts[slot], rows[slot], semgs[slot]
            col0 = g * jnp.int32(CHUNK)

            def grp(s, c):
                base = col0 + s * jnp.int32(LANES)
                rel = s * jnp.int32(LANES)
                a = []
                b = []
                u = []
                v = []
                for d in range(3):
                    xd = xv[jnp.int32(d), pl.ds(base, LANES)]
                    xx = (xd + 1.0) / 2.0 * RESOLUTION
                    xi = xx.astype(jnp.int32)
                    xf = xx - xi.astype(jnp.float32)
                    ad = xi * jnp.int32(PRIMES3[d])
                    a.append(ad)
                    b.append(ad + jnp.int32(PRIMES3[d]))
                    u.append(1.0 - xf)
                    v.append(xf)
                for k in range(8):
                    t0 = b[0] if k & 1 else a[0]
                    t1 = b[1] if k & 2 else a[1]
                    t2 = b[2] if k & 4 else a[2]
                    idxv[jnp.int32(k), pl.ds(rel, LANES)] = (
                        (t0 ^ t1 ^ t2) & jnp.int32(HASHMAP_SIZE - 1)
                    )
                    w0 = v[0] if k & 1 else u[0]
                    w1 = v[1] if k & 2 else u[1]
                    w2 = v[2] if k & 4 else u[2]
                    wgtv[jnp.int32(k), pl.ds(rel, LANES)] = w0 * w1 * w2
                return c

            lax.fori_loop(jnp.int32(0), jnp.int32(CHUNK // LANES), grp,
                          jnp.int32(0))
            for k in range(8):
                pltpu.async_copy(
                    w_hbm.at[idxv.at[jnp.int32(k)]],
                    rowsv.at[jnp.int32(k)],
                    semg,
                )

        def compute(slot, g, drain_out):
            idxv, wgtv, rowsv = idxs[slot], wgts[slot], rows[slot]
            outv, semg, semo = outs[slot], semgs[slot], semos[slot]
            ptb = pt0 + g * jnp.int32(CHUNK)
            for k in range(8):
                pltpu.make_async_copy(
                    w_hbm.at[idxv.at[jnp.int32(k)]],
                    rowsv.at[jnp.int32(k)],
                    semg,
                ).wait()

            @pl.when(drain_out)
            def _():
                pltpu.make_async_copy(
                    outv, out_hbm.at[pl.ds(ptb, CHUNK)], semo
                ).wait()

            def grp(s, c):
                for i in range(LANES):
                    p = s * jnp.int32(LANES) + jnp.int32(i)
                    pv = jnp.full((LANES,), p, jnp.int32)
                    wbc = [
                        plsc.load_gather(
                            wgtv,
                            [jnp.full((LANES,), jnp.int32(k), jnp.int32), pv],
                        )
                        for k in range(8)
                    ]
                    for fv in range(NV):
                        sl = pl.ds(fv * LANES, LANES)
                        acc = wbc[0] * rowsv[jnp.int32(0), p, sl]
                        for k in range(1, 8):
                            acc = acc + wbc[k] * rowsv[jnp.int32(k), p, sl]
                        outv[p, sl] = acc
                return c

            lax.fori_loop(jnp.int32(0), jnp.int32(CHUNK // LANES), grp,
                          jnp.int32(0))
            pltpu.async_copy(outv, out_hbm.at[pl.ds(ptb, CHUNK)], semo)

        stage(0, jnp.int32(0))

        def body(t, c):
            g0 = t * jnp.int32(2)
            stage(1, g0 + jnp.int32(1))
            compute(0, g0, g0 >= jnp.int32(2))

            @pl.when(g0 + jnp.int32(2) < jnp.int32(n_chunks))
            def _():
                stage(0, g0 + jnp.int32(2))

            compute(1, g0 + jnp.int32(1), g0 >= jnp.int32(1))
            return c

        lax.fori_loop(jnp.int32(0), jnp.int32(n_chunks // 2), body,
                      jnp.int32(0))

        last0 = pt0 + jnp.int32((n_chunks - 2) * CHUNK)
        last1 = pt0 + jnp.int32((n_chunks - 1) * CHUNK)
        pltpu.make_async_copy(
            outv0, out_hbm.at[pl.ds(last0, CHUNK)], semo0
        ).wait()
        pltpu.make_async_copy(
            outv1, out_hbm.at[pl.ds(last1, CHUNK)], semo1
        ).wait()

    return sc_kernel(xt, W)


def kernel(x, W):
    n = x.shape[0]
    xt = x.T.reshape(3 * n)
    return _sc_fused(xt, W, n)
